# Initial kernel scaffold; baseline (speedup 1.0000x reference)
import jax
import jax.numpy as jnp
from jax.experimental import pallas as pl


def kernel(pred_logits, init_reference, labels, boxes, start_ids, end_ids, masks, select_ids):
    raise NotImplementedError("write your pallas kernel here")


# speedup vs baseline: 55.2391x; 55.2391x over previous
import functools

import jax
import jax.numpy as jnp
from jax.experimental import pallas as pl

BS = 4
NUM_PATCH = 2
NQ = 2500
NGT = 64
K = 4
BG_LABEL = 400
THRESH = 0.6
NSEL = NGT // NUM_PATCH
MAXP = NUM_PATCH * NSEL * K


def _iota(shape, dim, dtype=jnp.int32):
    return jax.lax.broadcasted_iota(dtype, shape, dim)


def _col_of(row, n):
    eye = _iota((n, n), 0) == _iota((n, n), 1)
    return jnp.sum(jnp.where(eye, row, 0), axis=1, keepdims=True)


def _row_of(col, n):
    eye = _iota((n, n), 0) == _iota((n, n), 1)
    return jnp.sum(jnp.where(eye, col, 0), axis=0, keepdims=True)


def _assigner_body(prop_ref, boxt_ref, lab_ref, start_ref, end_ref, mask_ref,
                   sel_ref, iou_ref, x_ref, y_ref):
    props = prop_ref[0]
    boxt = boxt_ref[0]
    labels = lab_ref[0]
    starts = start_ref[0]
    ends = end_ref[0]
    gmask = mask_ref[0]
    sel = sel_ref[0]

    iota_g_row = _iota((1, NGT), 1)
    iota_g_col = _iota((NGT, NGT), 0)
    iota_s_col = _iota((NSEL, 1), 0)
    iota_m_col = _iota((NSEL, 1), 0)
    iota_q_row = _iota((1, NQ), 1)
    iota_k_row = _iota((1, K), 1)

    take_cols = []
    candx_cols = []
    candy_cols = []

    for p in range(NUM_PATCH):
        start_p = sel[p:p + 1, 0:1]
        end_p = sel[p:p + 1, 1:2]
        cat = (gmask != 0) & (starts == start_p) & (ends == end_p)
        cat_col = _col_of(cat.astype(jnp.int32), NGT) != 0
        tri = _iota((NGT, NGT), 0) < _iota((NGT, NGT), 1)
        rank0 = jnp.sum(jnp.where(tri & cat_col, 1, 0), axis=0,
                        keepdims=True)
        count = jnp.sum(cat.astype(jnp.int32), axis=1, keepdims=True)
        onehot = cat & (rank0 == iota_s_col)
        onehot = onehot | ((iota_s_col >= count) & (iota_g_row == 0))

        def pick(row):
            return jnp.sum(jnp.where(onehot, row, 0), axis=1, keepdims=True)

        sel_global = pick(iota_g_row)
        tgt_lab = pick(labels)
        tcx = pick(boxt[0:1, :])
        tcy = pick(boxt[1:2, :])
        tw = pick(boxt[2:3, :])
        th = pick(boxt[3:4, :])
        tx0, ty0 = tcx - 0.5 * tw, tcy - 0.5 * th
        tx1, ty1 = tcx + 0.5 * tw, tcy + 0.5 * th
        area1 = (tx1 - tx0) * (ty1 - ty0)

        off = p * NQ
        pcx = props[0:1, off:off + NQ]
        pcy = props[1:2, off:off + NQ]
        pw = props[2:3, off:off + NQ]
        ph = props[3:4, off:off + NQ]
        px0, py0 = pcx - 0.5 * pw, pcy - 0.5 * ph
        px1, py1 = pcx + 0.5 * pw, pcy + 0.5 * ph
        area2 = (px1 - px0) * (py1 - py0)

        iw = jnp.maximum(jnp.minimum(tx1, px1) - jnp.maximum(tx0, px0), 0.0)
        ih = jnp.maximum(jnp.minimum(ty1, py1) - jnp.maximum(ty0, py0), 0.0)
        inter = iw * ih
        union = area1 + area2 - inter
        iou = inter / union

        iou_ref[0, p * NSEL:(p + 1) * NSEL, :] = iou

        vals = jnp.max(iou, axis=0, keepdims=True)
        is_max = iou == vals
        matched = jnp.min(jnp.where(is_max, iota_m_col, NSEL), axis=0,
                          keepdims=True)
        highest = jnp.max(iou, axis=1, keepdims=True)
        lowq = jnp.sum(jnp.where(iou == highest, 1, 0), axis=0,
                       keepdims=True) > 0
        mlab = (vals >= THRESH) | lowq
        onehot_m = matched == iota_m_col
        bg_col = tgt_lab == float(BG_LABEL)
        matched_bg = jnp.sum(jnp.where(onehot_m & bg_col, 1, 0), axis=0,
                             keepdims=True) > 0
        pos = mlab & jnp.logical_not(matched_bg)
        counts = jnp.sum(jnp.where(onehot_m & pos, 1, 0), axis=1,
                         keepdims=True)
        take_cols.append(jnp.minimum(counts, K))

        work = iou
        for k in range(K):
            m = jnp.max(work, axis=1, keepdims=True)
            a = jnp.min(jnp.where(work == m, iota_q_row, NQ), axis=1,
                        keepdims=True)
            candx_cols.append(a + off)
            candy_cols.append(sel_global)
            work = jnp.where(iota_q_row == a, -1.0, work)

    take_col = jnp.concatenate(take_cols, axis=0)
    tri_g = _iota((NGT, NGT), 0) < _iota((NGT, NGT), 1)
    excl_row = jnp.sum(jnp.where(tri_g, jnp.broadcast_to(take_col,
                                                         (NGT, NGT)), 0),
                       axis=0, keepdims=True)
    excl_col = _col_of(excl_row, NGT)

    iota_o_row = _iota((1, MAXP), 1)
    acc_x = jnp.zeros((1, MAXP), jnp.int32)
    acc_y = jnp.zeros((1, MAXP), jnp.int32)
    hit = jnp.zeros((1, MAXP), jnp.int32)
    for k in range(K):
        cx = jnp.concatenate([candx_cols[p * K + k] for p in range(NUM_PATCH)],
                             axis=0)
        cy = jnp.concatenate([candy_cols[p * K + k] for p in range(NUM_PATCH)],
                             axis=0)
        valid = k < take_col
        oh = valid & (excl_col + k == iota_o_row)
        acc_x = acc_x + jnp.sum(jnp.where(oh, cx, 0), axis=0, keepdims=True)
        acc_y = acc_y + jnp.sum(jnp.where(oh, cy, 0), axis=0, keepdims=True)
        hit = hit + jnp.sum(jnp.where(oh, 1, 0), axis=0, keepdims=True)

    x_ref[0] = jnp.where(hit > 0, acc_x, -1)
    y_ref[0] = jnp.where(hit > 0, acc_y, -1)


@jax.jit
def _run(init_reference, labels, boxes, start_ids, end_ids, masks, select_ids):
    props = jnp.transpose(init_reference, (0, 2, 1))
    boxt = jnp.transpose(boxes, (0, 2, 1))
    lab = labels.astype(jnp.float32).reshape(BS, 1, NGT)
    st = start_ids.astype(jnp.int32).reshape(BS, 1, NGT)
    en = end_ids.astype(jnp.int32).reshape(BS, 1, NGT)
    mk = masks.astype(jnp.int32).reshape(BS, 1, NGT)
    sel = select_ids.astype(jnp.int32)

    ious, xr, yr = pl.pallas_call(
        _assigner_body,
        grid=(BS,),
        in_specs=[
            pl.BlockSpec((1, 4, NUM_PATCH * NQ), lambda b: (b, 0, 0)),
            pl.BlockSpec((1, 4, NGT), lambda b: (b, 0, 0)),
            pl.BlockSpec((1, 1, NGT), lambda b: (b, 0, 0)),
            pl.BlockSpec((1, 1, NGT), lambda b: (b, 0, 0)),
            pl.BlockSpec((1, 1, NGT), lambda b: (b, 0, 0)),
            pl.BlockSpec((1, 1, NGT), lambda b: (b, 0, 0)),
            pl.BlockSpec((1, NUM_PATCH, 2), lambda b: (b, 0, 0)),
        ],
        out_specs=[
            pl.BlockSpec((1, NUM_PATCH * NSEL, NQ), lambda b: (b, 0, 0)),
            pl.BlockSpec((1, 1, MAXP), lambda b: (b, 0, 0)),
            pl.BlockSpec((1, 1, MAXP), lambda b: (b, 0, 0)),
        ],
        out_shape=[
            jax.ShapeDtypeStruct((BS, NUM_PATCH * NSEL, NQ), jnp.float32),
            jax.ShapeDtypeStruct((BS, 1, MAXP), jnp.int32),
            jax.ShapeDtypeStruct((BS, 1, MAXP), jnp.int32),
        ],
    )(props, boxt, lab, st, en, mk, sel)
    return ious, xr.reshape(BS, MAXP), yr.reshape(BS, MAXP)


def kernel(pred_logits, init_reference, labels, boxes, start_ids, end_ids,
           masks, select_ids):
    del pred_logits
    return _run(init_reference, labels, boxes, start_ids, end_ids, masks,
                select_ids)
